# baseline (speedup 1.0000x reference)
import functools

import jax
import jax.numpy as jnp
from jax.experimental import pallas as pl

_MASK_FRAC = 0.5
_S = 16
_H = 224


def _interp_matrix():
    return jax.image.resize(jnp.eye(_S, dtype=jnp.float32), (_H, _S),
                            method="bilinear")


def _plane_kernel(x_ref, m_ref, a_ref, o_ref):
    a = a_ref[...]
    m = m_ref[0]
    t = jnp.dot(a, m, preferred_element_type=jnp.float32,
                precision=jax.lax.Precision.HIGHEST)
    mu = jax.lax.dot_general(t, a, (((1,), (1,)), ((), ())),
                             preferred_element_type=jnp.float32,
                             precision=jax.lax.Precision.HIGHEST)
    o_ref[0] = jnp.where(mu < _MASK_FRAC, 0.0, x_ref[0])


@functools.partial(jax.jit, static_argnames=("interpret",))
def _run(x, mask, interpret=False):
    B, C, H, W = x.shape
    xp = x.reshape(B * C, H, W)
    mp = mask.reshape(B * C, _S, _S)
    a = _interp_matrix()
    out = pl.pallas_call(
        _plane_kernel,
        grid=(B * C,),
        in_specs=[
            pl.BlockSpec((1, H, W), lambda i: (i, 0, 0)),
            pl.BlockSpec((1, _S, _S), lambda i: (i, 0, 0)),
            pl.BlockSpec((_H, _S), lambda i: (0, 0)),
        ],
        out_specs=pl.BlockSpec((1, H, W), lambda i: (i, 0, 0)),
        out_shape=jax.ShapeDtypeStruct((B * C, H, W), jnp.float32),
        interpret=interpret,
    )(xp, mp, a)
    return out.reshape(B, C, H, W)


def kernel(x, mask):
    return _run(x, mask)
